# baseline (speedup 1.0000x reference)
import functools

import jax
import jax.numpy as jnp
from jax import lax
from jax.experimental import pallas as pl
from jax.experimental.pallas import tpu as pltpu
from jax.experimental.pallas import tpu_sc as plsc

N_NODES = 10000
DEG = 32
D_FEAT = 128
L = 16
NJ = D_FEAT // L

NSC = 400
NTC = N_NODES - NSC
TC_BLOCK = 400

NB = 8
NHALF = 4
SC_CHUNKS = NSC // NB
NC, NS = 2, 16
NW = NC * NS
K = 2 * ((SC_CHUNKS + 2 * NW - 1) // (2 * NW))

_mesh = plsc.VectorSubcoreMesh(
    core_axis_name="c", subcore_axis_name="s", num_cores=NC, num_subcores=NS
)


@functools.partial(
    pl.kernel,
    out_type=jax.ShapeDtypeStruct((NSC, D_FEAT), jnp.float32),
    mesh=_mesh,
    scratch_types=[
        pltpu.VMEM((NB, DEG, D_FEAT), jnp.float32),
        pltpu.VMEM((NB, DEG, D_FEAT), jnp.float32),
        pltpu.VMEM((NB, D_FEAT), jnp.float32),
        pltpu.VMEM((NB, D_FEAT), jnp.float32),
        pltpu.SemaphoreType.DMA,
        pltpu.SemaphoreType.DMA,
        pltpu.SemaphoreType.DMA,
        pltpu.SemaphoreType.DMA,
    ],
)
def _sc_mean(x_hbm, o_hbm, buf0, buf1, acc0, acc1, ls0, ls1, os0, os1):
    wid = lax.axis_index("c") * NS + lax.axis_index("s")

    def chunk_of(k):
        return jnp.minimum(wid + k * NW, SC_CHUNKS - 1)

    def start_load(k, buf, sem):
        t = chunk_of(k)
        pltpu.make_async_copy(
            x_hbm.at[pl.ds(t * NB, NB)], buf, sem
        ).start()

    def wait_load(buf, sem):
        pltpu.make_async_copy(x_hbm.at[pl.ds(0, NB)], buf, sem).wait()

    def start_store(k, acc, sem):
        t = chunk_of(k)
        pltpu.make_async_copy(acc, o_hbm.at[pl.ds(t * NB, NB)], sem).start()

    def wait_store(acc, sem):
        pltpu.make_async_copy(acc, o_hbm.at[pl.ds(0, NB)], sem).wait()

    def compute(buf, acc):
        for h in range(NB // NHALF):
            def dbody(d, accs):
                return tuple(
                    tuple(
                        accs[n][j] + buf[h * NHALF + n, d, pl.ds(j * L, L)]
                        for j in range(NJ)
                    )
                    for n in range(NHALF)
                )

            init = tuple(
                tuple(buf[h * NHALF + n, 0, pl.ds(j * L, L)] for j in range(NJ))
                for n in range(NHALF)
            )
            accs = lax.fori_loop(1, DEG, dbody, init, unroll=2)
            for n in range(NHALF):
                for j in range(NJ):
                    acc[h * NHALF + n, pl.ds(j * L, L)] = accs[n][j] * (
                        1.0 / DEG
                    )

    start_load(0, buf0, ls0)
    start_load(1, buf1, ls1)

    def outer(j, carry):
        for b, buf, acc, lsem, osem in (
            (0, buf0, acc0, ls0, os0),
            (1, buf1, acc1, ls1, os1),
        ):
            k = 2 * j + b
            wait_load(buf, lsem)

            @pl.when(j >= 1)
            def _():
                wait_store(acc, osem)

            compute(buf, acc)

            @pl.when(j < K // 2 - 1)
            def _():
                start_load(k + 2, buf, lsem)

            start_store(k, acc, osem)
        return carry

    lax.fori_loop(0, K // 2, outer, 0, unroll=False)

    wait_store(acc0, os0)
    wait_store(acc1, os1)


def _tc_mean(x_ref, o_ref):
    o_ref[...] = jnp.sum(x_ref[...], axis=1) * (1.0 / DEG)


def kernel(neighbour):
    out_sc = _sc_mean(lax.slice_in_dim(neighbour, NTC, N_NODES, axis=0))
    out_tc = pl.pallas_call(
        _tc_mean,
        grid=(NTC // TC_BLOCK,),
        in_specs=[pl.BlockSpec((TC_BLOCK, DEG, D_FEAT), lambda i: (i, 0, 0))],
        out_specs=pl.BlockSpec((TC_BLOCK, D_FEAT), lambda i: (i, 0)),
        out_shape=jax.ShapeDtypeStruct((NTC, D_FEAT), jnp.float32),
    )(neighbour)
    return jnp.concatenate([out_tc, out_sc], axis=0)


# speedup vs baseline: 1.5142x; 1.5142x over previous
import jax
import jax.numpy as jnp
from jax.experimental import pallas as pl

N_NODES = 10000
DEG = 32
D_FEAT = 128
BLOCK = 400


def _mean_kernel(a_ref, b_ref, o_ref):
    o_ref[...] = (
        jnp.sum(a_ref[...], axis=1) + jnp.sum(b_ref[...], axis=1)
    ) * (1.0 / DEG)


def kernel(neighbour):
    return pl.pallas_call(
        _mean_kernel,
        grid=(N_NODES // BLOCK,),
        in_specs=[
            pl.BlockSpec((BLOCK, DEG // 2, D_FEAT), lambda i: (i, 0, 0)),
            pl.BlockSpec((BLOCK, DEG // 2, D_FEAT), lambda i: (i, 1, 0)),
        ],
        out_specs=pl.BlockSpec((BLOCK, D_FEAT), lambda i: (i, 0)),
        out_shape=jax.ShapeDtypeStruct((N_NODES, D_FEAT), jnp.float32),
    )(neighbour, neighbour)


# speedup vs baseline: 1.5696x; 1.0366x over previous
import jax
import jax.numpy as jnp
from jax.experimental import pallas as pl

N_NODES = 10000
DEG = 32
D_FEAT = 128
BLOCK = 400


def _mean_kernel(x_ref, o_ref):
    o_ref[...] = jnp.sum(x_ref[...], axis=1) * (1.0 / DEG)


def kernel(neighbour):
    return pl.pallas_call(
        _mean_kernel,
        grid=(N_NODES // BLOCK,),
        in_specs=[pl.BlockSpec((BLOCK, DEG, D_FEAT), lambda i: (i, 0, 0))],
        out_specs=pl.BlockSpec((BLOCK, D_FEAT), lambda i: (i, 0)),
        out_shape=jax.ShapeDtypeStruct((N_NODES, D_FEAT), jnp.float32),
    )(neighbour)
